# baseline (speedup 1.0000x reference)
import functools

import jax
import jax.numpy as jnp
from jax import lax
from jax.experimental import pallas as pl
from jax.experimental.pallas import tpu as pltpu
from jax.experimental.pallas import tpu_sc as plsc

N = 10000
E = 320000
D = 128

NC = 2
NS = 16
NW = NC * NS
RANGE = 8 * (-(-N // (8 * NW)))
LAST = N - (NW - 1) * RANGE

CE = 3200
FB = 2048
CAP = E + 2 * FB
GC = 128
SC2 = 2048

_mesh = plsc.VectorSubcoreMesh(core_axis_name="c", subcore_axis_name="s")


def _wid():
    return lax.axis_index("s") * NC + lax.axis_index("c")


@functools.partial(
    pl.kernel,
    mesh=_mesh,
    out_type=[
        jax.ShapeDtypeStruct((NW, CAP), jnp.int32),
        jax.ShapeDtypeStruct((NW, CAP), jnp.int32),
        jax.ShapeDtypeStruct((NW, 16), jnp.int32),
        jax.ShapeDtypeStruct((NW * CAP,), jnp.int32),
        jax.ShapeDtypeStruct((NW * CAP,), jnp.int32),
    ],
    scratch_types=[
        pltpu.VMEM((CE,), jnp.int32),
        pltpu.VMEM((CE,), jnp.int32),
        pltpu.VMEM((2 * FB,), jnp.int32),
        pltpu.VMEM((2 * FB,), jnp.int32),
        pltpu.VMEM(((RANGE + 1) * 16,), jnp.int32),
        pltpu.VMEM((SC2,), jnp.int32),
        pltpu.VMEM((SC2,), jnp.int32),
        pltpu.VMEM((16, 128), jnp.int32),
        pltpu.VMEM((16,), jnp.int32),
        pltpu.SemaphoreType.DMA,
    ],
    compiler_params=pltpu.CompilerParams(needs_layout_passes=False),
)
def _plan(ei, src_list, dloc_list, counts, ssrc, sdlo,
          src_b, dst_b, csrc, cdlo, hist, dbuf, sbuf, posb, cnt_v, sem_s):
    w = _wid()
    lo = w * RANGE
    hi = jnp.minimum(lo + RANGE, N)
    lane = lax.broadcasted_iota(jnp.int32, (16,), 0)

    def chunk_body(ci, carry):
        pltpu.sync_copy(ei.at[0, pl.ds(ci * CE, CE)], src_b)
        pltpu.sync_copy(ei.at[1, pl.ds(ci * CE, CE)], dst_b)

        def vbody(k, c2):
            cnt, nfl = c2
            d = dst_b[pl.ds(k * 16, 16)]
            s = src_b[pl.ds(k * 16, 16)]
            m = (d >= lo) & (d < hi)
            pos = plsc.cumsum(m.astype(jnp.int32))
            addr = cnt + pos - 1
            plsc.store_scatter(csrc, [addr], s, mask=m)
            plsc.store_scatter(cdlo, [addr], d - lo, mask=m)
            cnt = cnt + jnp.max(pos)
            do = cnt >= FB

            @pl.when(do)
            def _():
                pltpu.sync_copy(csrc.at[pl.ds(0, FB)],
                                src_list.at[w, pl.ds(nfl * FB, FB)])
                pltpu.sync_copy(cdlo.at[pl.ds(0, FB)],
                                dloc_list.at[w, pl.ds(nfl * FB, FB)])
                ts = csrc[pl.ds(FB, 16)]
                td = cdlo[pl.ds(FB, 16)]
                csrc[pl.ds(0, 16)] = ts
                cdlo[pl.ds(0, 16)] = td

            cnt = jnp.where(do, cnt - FB, cnt)
            nfl = jnp.where(do, nfl + 1, nfl)
            return (cnt, nfl)

        return lax.fori_loop(0, CE // 16, vbody, carry)

    cnt, nfl = lax.fori_loop(0, E // CE, chunk_body,
                             (jnp.int32(0), jnp.int32(0)))

    pad_s = w * 97 + lane * 13
    pad_d = jnp.full((16,), RANGE, jnp.int32)
    for t in range(GC // 16):
        csrc[pl.ds(cnt + t * 16, 16)] = pad_s
        cdlo[pl.ds(cnt + t * 16, 16)] = pad_d

    pltpu.sync_copy(csrc.at[pl.ds(0, 2 * FB)],
                    src_list.at[w, pl.ds(nfl * FB, 2 * FB)])
    pltpu.sync_copy(cdlo.at[pl.ds(0, 2 * FB)],
                    dloc_list.at[w, pl.ds(nfl * FB, 2 * FB)])
    total = nfl * FB + cnt
    nch = lax.shift_right_logical(total + (GC - 1), 7)
    cnt_v[...] = jnp.broadcast_to(nch, (16,))
    pltpu.sync_copy(cnt_v, counts.at[w])

    one_v = jnp.full((16,), 1, jnp.int32)
    zero_v = jnp.zeros((16,), jnp.int32)
    nvr_total = nch * (GC // 16)

    def clr(i, _):
        hist[pl.ds(i * 16, 16)] = zero_v
        return 0

    lax.fori_loop(0, RANGE + 1, clr, 0)

    nsc = lax.shift_right_logical(nch * GC + (SC2 - 1), 11)

    def hchunk(c, _):
        pltpu.sync_copy(dloc_list.at[w, pl.ds(c * SC2, SC2)], dbuf)
        nv = jnp.minimum(SC2 // 16, nvr_total - c * (SC2 // 16))

        def hv(k, _):
            d = dbuf[pl.ds(k * 16, 16)]
            plsc.addupdate_scatter(hist, [d * 16 + lane], one_v)
            return 0

        lax.fori_loop(0, nv, hv, 0)
        return 0

    lax.fori_loop(0, nsc, hchunk, 0)

    def ob(b, run):
        row = hist[pl.ds(b * 16, 16)]
        cs = plsc.cumsum(row)
        hist[pl.ds(b * 16, 16)] = run + (cs - row)
        return run + jnp.max(cs)

    lax.fori_loop(0, RANGE + 1, ob, lax.mul(w, CAP))

    def pchunk(c, _):
        pltpu.sync_copy(dloc_list.at[w, pl.ds(c * SC2, SC2)], dbuf)
        pltpu.sync_copy(src_list.at[w, pl.ds(c * SC2, SC2)], sbuf)
        nv = jnp.minimum(SC2 // 16, nvr_total - c * (SC2 // 16))

        def pv(k, _):
            d = dbuf[pl.ds(k * 16, 16)]
            a = d * 16 + lane
            p = plsc.load_gather(hist, [a])
            plsc.store_scatter(hist, [a], p + 1)
            posb[lax.shift_right_logical(k, 3),
                 pl.ds(lax.mul(jnp.bitwise_and(k, 7), 16), 16)] = p
            return 0

        lax.fori_loop(0, nv, pv, 0)
        ng = lax.shift_right_logical(nv + 7, 3)

        def sg(r, _):
            pltpu.async_copy(sbuf.at[pl.ds(r * 128, 128)],
                             ssrc.at[posb.at[r]], sem_s)
            pltpu.async_copy(dbuf.at[pl.ds(r * 128, 128)],
                             sdlo.at[posb.at[r]], sem_s)
            return 0

        lax.fori_loop(0, ng, sg, 0)

        def dr(r, _):
            pltpu.make_async_copy(sbuf.at[pl.ds(r * 128, 128)],
                                  ssrc.at[posb.at[r]], sem_s).wait()
            pltpu.make_async_copy(dbuf.at[pl.ds(r * 128, 128)],
                                  sdlo.at[posb.at[r]], sem_s).wait()
            return 0

        lax.fori_loop(0, ng, dr, 0)
        return 0

    lax.fori_loop(0, nsc, pchunk, 0)


@functools.partial(
    pl.kernel,
    mesh=_mesh,
    out_type=jax.ShapeDtypeStruct((N, D), jnp.float32),
    scratch_types=[
        [pltpu.VMEM(((RANGE + 1) * 16,), jnp.float32)
         for _ in range(D // 16)],
        pltpu.VMEM((GC,), jnp.int32),
        pltpu.VMEM((GC,), jnp.int32),
        pltpu.VMEM((GC,), jnp.int32),
        pltpu.VMEM((GC,), jnp.int32),
        pltpu.VMEM((GC, D), jnp.float32),
        pltpu.VMEM((GC, D), jnp.float32),
        pltpu.VMEM((40, D), jnp.float32),
        pltpu.VMEM((16,), jnp.int32),
        pltpu.SemaphoreType.DMA,
        pltpu.SemaphoreType.DMA,
    ],
    compiler_params=pltpu.CompilerParams(needs_layout_passes=False),
)
def _aggregate(h, ssrc, sdlo, counts, out,
               aggr8, idx0, idx1, dl0, dl1, rows0, rows1, merged, cnt_v,
               sem0, sem1):
    w = _wid()
    lo = pl.multiple_of(w * RANGE, 8)
    base = pl.multiple_of(lax.mul(w, CAP), 128)

    pltpu.sync_copy(counts.at[w], cnt_v)
    nch = jnp.max(cnt_v[...])

    zero_v = jnp.zeros((16,), jnp.float32)

    def init_body(i, _):
        for a in aggr8:
            a[pl.ds(i * 16, 16)] = zero_v
        return 0

    lax.fori_loop(0, RANGE + 1, init_body, 0)

    @pl.when(nch > 0)
    def _():
        pltpu.sync_copy(ssrc.at[pl.ds(base, GC)], idx0)
        pltpu.sync_copy(sdlo.at[pl.ds(base, GC)], dl0)
        pltpu.async_copy(h.at[idx0], rows0, sem0)

    lane = lax.broadcasted_iota(jnp.int32, (16,), 0)
    neg_inf = jnp.full((16,), -jnp.inf, jnp.float32)

    bufs = ((idx0, dl0, rows0, sem0), (idx1, dl1, rows1, sem1))

    def compute(dl_b, rows_b, carry):
        def qbody(q, carry):
            prev = carry[0]
            acc = list(carry[1:])
            for e in range(16):
                row = q * 16 + e
                dle = plsc.load_gather(
                    dl_b, [jnp.full((16,), row, jnp.int32)])
                addr = dle * 16 + lane
                fresh = dle != prev
                for j in range(D // 16):
                    msg = rows_b[row, pl.ds(j * 16, 16)]
                    acc[j] = jnp.where(fresh, msg,
                                       jnp.maximum(acc[j], msg))
                    plsc.store_scatter(aggr8[j], [addr], acc[j])
                prev = dle
            return (prev, *acc)

        return lax.fori_loop(0, GC // 16, qbody, carry)

    def body(g, carry):
        def work(bc, bn, carry):
            idx_c, dl_c, rows_c, sem_c = bc
            idx_n, dl_n, rows_n, sem_n = bn

            @pl.when(g + 1 < nch)
            def _():
                pltpu.sync_copy(ssrc.at[pl.ds(base + (g + 1) * GC, GC)],
                                idx_n)
                pltpu.sync_copy(sdlo.at[pl.ds(base + (g + 1) * GC, GC)],
                                dl_n)
                pltpu.async_copy(h.at[idx_n], rows_n, sem_n)

            pltpu.make_async_copy(h.at[idx_c], rows_c, sem_c).wait()
            return compute(dl_c, rows_c, carry)

        return lax.cond(jnp.bitwise_and(g, 1) == 0,
                        lambda c: work(bufs[0], bufs[1], c),
                        lambda c: work(bufs[1], bufs[0], c),
                        carry)

    carry0 = (jnp.full((16,), -1, jnp.int32),) + (neg_inf,) * (D // 16)
    lax.fori_loop(0, nch, body, carry0)

    nmb = jnp.where(w == NW - 1, LAST // 40, RANGE // 40)

    def chunk_out(c, _):
        def fin_body(i, _):
            for j, a in enumerate(aggr8):
                merged[i, pl.ds(j * 16, 16)] = a[
                    pl.ds((c * 40 + i) * 16, 16)]
            return 0

        lax.fori_loop(0, 40, fin_body, 0)
        pltpu.sync_copy(merged, out.at[pl.ds(lo + c * 40, 40)])
        return 0

    lax.fori_loop(0, nmb, chunk_out, 0)


def _pre_body(x_ref, w_ref, b_ref, o_ref):
    o_ref[...] = (
        jnp.dot(x_ref[...], w_ref[...], preferred_element_type=jnp.float32)
        + b_ref[...]
    )


def _conv_body(act, a_ref, h_ref, wl_ref, wr_ref, b_ref, o_ref):
    y = (
        jnp.dot(a_ref[...], wl_ref[...], preferred_element_type=jnp.float32)
        + jnp.dot(h_ref[...], wr_ref[...], preferred_element_type=jnp.float32)
        + b_ref[...]
    )
    if act == "relu":
        y = jnp.maximum(y, 0.0)
    elif act == "l2":
        n = jnp.sqrt(jnp.sum(y * y, axis=-1, keepdims=True))
        y = y / jnp.maximum(n, 1e-12)
    o_ref[...] = y


def _pre(x, wt, b):
    return pl.pallas_call(
        _pre_body,
        out_shape=jax.ShapeDtypeStruct((N, D), jnp.float32),
    )(x, wt, b.reshape(1, D))


def _conv(act, aggr, h, wlt, wrt, b):
    return pl.pallas_call(
        functools.partial(_conv_body, act),
        out_shape=jax.ShapeDtypeStruct((N, D), jnp.float32),
    )(aggr, h, wlt, wrt, b.reshape(1, D))


def kernel(x, edge_index, W_pre, b_pre, W_l_first, b_l_first, W_r_first,
           W_l_h0, b_l_h0, W_r_h0, W_l_h1, b_l_h1, W_r_h1,
           W_l_h2, b_l_h2, W_r_h2, W_l_out, b_l_out, W_r_out):
    _, _, counts, ssrc, sdlo = _plan(edge_index)

    wp = W_pre.T
    wlf, wrf = W_l_first.T, W_r_first.T
    wlo, wro = W_l_out.T, W_r_out.T
    hidden = [(W_l_h0.T, b_l_h0, W_r_h0.T),
              (W_l_h1.T, b_l_h1, W_r_h1.T),
              (W_l_h2.T, b_l_h2, W_r_h2.T)]

    outs = []
    h = x
    for i in range(3):
        h = _pre(h, wp, b_pre)
        a = _aggregate(h, ssrc, sdlo, counts)
        h = _conv("relu", a, h, wlf, wrf, b_l_first)
        wlh, blh, wrh = hidden[i]
        a = _aggregate(h, ssrc, sdlo, counts)
        h = _conv("relu", a, h, wlh, wrh, blh)
        a = _aggregate(h, ssrc, sdlo, counts)
        h = _conv("l2", a, h, wlo, wro, b_l_out)
        outs.append(h)
    return tuple(outs)


# speedup vs baseline: 2.1213x; 2.1213x over previous
import functools

import jax
import jax.numpy as jnp
from jax import lax
from jax.experimental import pallas as pl
from jax.experimental.pallas import tpu as pltpu
from jax.experimental.pallas import tpu_sc as plsc

N = 10000
E = 320000
D = 128

NC = 2
NS = 16
NW = NC * NS
RANGE = 8 * (-(-N // (8 * NW)))
LAST = N - (NW - 1) * RANGE

CE = 3200
FB = 2048
CAP = E + 2 * FB
GC = 128
SC2 = 2048

_mesh = plsc.VectorSubcoreMesh(core_axis_name="c", subcore_axis_name="s")


def _wid():
    return lax.axis_index("s") * NC + lax.axis_index("c")


@functools.partial(
    pl.kernel,
    mesh=_mesh,
    out_type=[
        jax.ShapeDtypeStruct((NW, CAP), jnp.int32),
        jax.ShapeDtypeStruct((NW, CAP), jnp.int32),
        jax.ShapeDtypeStruct((NW, 16), jnp.int32),
        jax.ShapeDtypeStruct((NW * CAP,), jnp.int32),
        jax.ShapeDtypeStruct((NW * CAP,), jnp.int32),
    ],
    scratch_types=[
        pltpu.VMEM((CE,), jnp.int32),
        pltpu.VMEM((CE,), jnp.int32),
        pltpu.VMEM((2 * FB,), jnp.int32),
        pltpu.VMEM((2 * FB,), jnp.int32),
        pltpu.VMEM(((RANGE + 1) * 16,), jnp.int32),
        pltpu.VMEM((SC2,), jnp.int32),
        pltpu.VMEM((SC2,), jnp.int32),
        pltpu.VMEM((16, 128), jnp.int32),
        pltpu.VMEM((16,), jnp.int32),
        pltpu.VMEM((16, 16), jnp.int32),
        pltpu.VMEM_SHARED((16, 16), jnp.int32),
        pltpu.VMEM_SHARED((CAP,), jnp.int32),
        pltpu.VMEM_SHARED((CAP,), jnp.int32),
        pltpu.SemaphoreType.DMA,
    ],
    compiler_params=pltpu.CompilerParams(needs_layout_passes=False),
)
def _plan(ei, src_list, dloc_list, counts, ssrc, sdlo,
          src_b, dst_b, csrc, cdlo, hist, dbuf, sbuf, posb, cnt_v,
          lcl, shcnt, sh_src, sh_dlo, sem_s):
    w = _wid()
    lo = w * RANGE
    hi = jnp.minimum(lo + RANGE, N)
    lane = lax.broadcasted_iota(jnp.int32, (16,), 0)

    def chunk_body(ci, carry):
        pltpu.sync_copy(ei.at[0, pl.ds(ci * CE, CE)], src_b)
        pltpu.sync_copy(ei.at[1, pl.ds(ci * CE, CE)], dst_b)

        def vbody(k, c2):
            cnt, nfl = c2
            d = dst_b[pl.ds(k * 16, 16)]
            s = src_b[pl.ds(k * 16, 16)]
            m = (d >= lo) & (d < hi)
            pos = plsc.cumsum(m.astype(jnp.int32))
            addr = cnt + pos - 1
            plsc.store_scatter(csrc, [addr], s, mask=m)
            plsc.store_scatter(cdlo, [addr], d - lo, mask=m)
            cnt = cnt + jnp.max(pos)
            do = cnt >= FB

            @pl.when(do)
            def _():
                pltpu.sync_copy(csrc.at[pl.ds(0, FB)],
                                src_list.at[w, pl.ds(nfl * FB, FB)])
                pltpu.sync_copy(cdlo.at[pl.ds(0, FB)],
                                dloc_list.at[w, pl.ds(nfl * FB, FB)])
                ts = csrc[pl.ds(FB, 16)]
                td = cdlo[pl.ds(FB, 16)]
                csrc[pl.ds(0, 16)] = ts
                cdlo[pl.ds(0, 16)] = td

            cnt = jnp.where(do, cnt - FB, cnt)
            nfl = jnp.where(do, nfl + 1, nfl)
            return (cnt, nfl)

        return lax.fori_loop(0, CE // 16, vbody, carry)

    cnt, nfl = lax.fori_loop(0, E // CE, chunk_body,
                             (jnp.int32(0), jnp.int32(0)))

    pad_s = w * 97 + lane * 13
    pad_d = jnp.full((16,), RANGE, jnp.int32)
    for t in range(GC // 16):
        csrc[pl.ds(cnt + t * 16, 16)] = pad_s
        cdlo[pl.ds(cnt + t * 16, 16)] = pad_d

    pltpu.sync_copy(csrc.at[pl.ds(0, 2 * FB)],
                    src_list.at[w, pl.ds(nfl * FB, 2 * FB)])
    pltpu.sync_copy(cdlo.at[pl.ds(0, 2 * FB)],
                    dloc_list.at[w, pl.ds(nfl * FB, 2 * FB)])
    total = nfl * FB + cnt
    nch = lax.shift_right_logical(total + (GC - 1), 7)
    cnt_v[...] = jnp.broadcast_to(nch, (16,))
    pltpu.sync_copy(cnt_v, counts.at[w])

    sid = lax.axis_index("s")
    my_len = nch * GC
    cnt_v[...] = jnp.broadcast_to(my_len, (16,))
    pltpu.sync_copy(cnt_v, shcnt.at[sid])
    plsc.subcore_barrier()
    pltpu.sync_copy(shcnt, lcl)

    def pfx(r, b):
        v = jnp.max(lcl[r])
        return b + jnp.where(r < sid, v, 0)

    sbase = pl.multiple_of(lax.fori_loop(0, 16, pfx, jnp.int32(0)), 128)

    one_v = jnp.full((16,), 1, jnp.int32)
    zero_v = jnp.zeros((16,), jnp.int32)
    nvr_total = nch * (GC // 16)

    def clr(i, _):
        hist[pl.ds(i * 16, 16)] = zero_v
        return 0

    lax.fori_loop(0, RANGE + 1, clr, 0)

    nsc = lax.shift_right_logical(nch * GC + (SC2 - 1), 11)

    def hchunk(c, _):
        pltpu.sync_copy(dloc_list.at[w, pl.ds(c * SC2, SC2)], dbuf)
        nv = jnp.minimum(SC2 // 16, nvr_total - c * (SC2 // 16))

        def hv(k, _):
            d = dbuf[pl.ds(k * 16, 16)]
            plsc.addupdate_scatter(hist, [d * 16 + lane], one_v)
            return 0

        lax.fori_loop(0, nv, hv, 0)
        return 0

    lax.fori_loop(0, nsc, hchunk, 0)

    def ob(b, run):
        row = hist[pl.ds(b * 16, 16)]
        cs = plsc.cumsum(row)
        hist[pl.ds(b * 16, 16)] = run + (cs - row)
        return run + jnp.max(cs)

    lax.fori_loop(0, RANGE + 1, ob, sbase)

    def pchunk(c, _):
        pltpu.sync_copy(dloc_list.at[w, pl.ds(c * SC2, SC2)], dbuf)
        pltpu.sync_copy(src_list.at[w, pl.ds(c * SC2, SC2)], sbuf)
        nv = jnp.minimum(SC2 // 16, nvr_total - c * (SC2 // 16))

        def pv(k, _):
            d = dbuf[pl.ds(k * 16, 16)]
            a = d * 16 + lane
            p = plsc.load_gather(hist, [a])
            plsc.store_scatter(hist, [a], p + 1)
            posb[lax.shift_right_logical(k, 3),
                 pl.ds(lax.mul(jnp.bitwise_and(k, 7), 16), 16)] = p
            return 0

        lax.fori_loop(0, nv, pv, 0)
        ng = lax.shift_right_logical(nv + 7, 3)

        def sg(r, _):
            pltpu.async_copy(sbuf.at[pl.ds(r * 128, 128)],
                             sh_src.at[posb.at[r]], sem_s)
            pltpu.async_copy(dbuf.at[pl.ds(r * 128, 128)],
                             sh_dlo.at[posb.at[r]], sem_s)
            return 0

        lax.fori_loop(0, ng, sg, 0)

        def dr(r, _):
            pltpu.make_async_copy(sbuf.at[pl.ds(r * 128, 128)],
                                  sh_src.at[posb.at[r]], sem_s).wait()
            pltpu.make_async_copy(dbuf.at[pl.ds(r * 128, 128)],
                                  sh_dlo.at[posb.at[r]], sem_s).wait()
            return 0

        lax.fori_loop(0, ng, dr, 0)
        return 0

    lax.fori_loop(0, nsc, pchunk, 0)

    nblk = lax.shift_right_logical(my_len + (SC2 - 1), 11)
    hbase = pl.multiple_of(lax.mul(w, CAP), 128)

    def outb(bk, _):
        pltpu.sync_copy(sh_src.at[pl.ds(sbase + bk * SC2, SC2)],
                        ssrc.at[pl.ds(hbase + bk * SC2, SC2)])
        pltpu.sync_copy(sh_dlo.at[pl.ds(sbase + bk * SC2, SC2)],
                        sdlo.at[pl.ds(hbase + bk * SC2, SC2)])
        return 0

    lax.fori_loop(0, nblk, outb, 0)


@functools.partial(
    pl.kernel,
    mesh=_mesh,
    out_type=jax.ShapeDtypeStruct((N, D), jnp.float32),
    scratch_types=[
        [pltpu.VMEM(((RANGE + 1) * 16,), jnp.float32)
         for _ in range(D // 16)],
        pltpu.VMEM((GC,), jnp.int32),
        pltpu.VMEM((GC,), jnp.int32),
        pltpu.VMEM((GC + 16,), jnp.int32),
        pltpu.VMEM((GC + 16,), jnp.int32),
        pltpu.VMEM((GC, D), jnp.float32),
        pltpu.VMEM((GC, D), jnp.float32),
        pltpu.VMEM((40, D), jnp.float32),
        pltpu.VMEM((16,), jnp.int32),
        pltpu.SemaphoreType.DMA,
        pltpu.SemaphoreType.DMA,
    ],
    compiler_params=pltpu.CompilerParams(needs_layout_passes=False),
)
def _aggregate(h, ssrc, sdlo, counts, out,
               aggr8, idx0, idx1, dl0, dl1, rows0, rows1, merged, cnt_v,
               sem0, sem1):
    w = _wid()
    lo = pl.multiple_of(w * RANGE, 8)
    base = pl.multiple_of(lax.mul(w, CAP), 128)

    pltpu.sync_copy(counts.at[w], cnt_v)
    nch = jnp.max(cnt_v[...])

    zero_v = jnp.zeros((16,), jnp.float32)

    def init_body(i, _):
        for a in aggr8:
            a[pl.ds(i * 16, 16)] = zero_v
        return 0

    lax.fori_loop(0, RANGE + 1, init_body, 0)

    @pl.when(nch > 0)
    def _():
        pltpu.sync_copy(ssrc.at[pl.ds(base, GC)], idx0)
        pltpu.sync_copy(sdlo.at[pl.ds(base, GC)], dl0.at[pl.ds(0, GC)])
        pltpu.async_copy(h.at[idx0], rows0, sem0)

    lane = lax.broadcasted_iota(jnp.int32, (16,), 0)
    neg_inf = jnp.full((16,), -jnp.inf, jnp.float32)

    bufs = ((idx0, dl0, rows0, sem0), (idx1, dl1, rows1, sem1))

    def compute(dl_c, rows_b, ok_next, dl_n, carry):
        okv = jnp.broadcast_to(ok_next.astype(jnp.int32), (16,))
        dl_c[pl.ds(GC, 16)] = dl_n[pl.ds(0, 16)] * okv + (okv - 1)

        def qbody(q, carry):
            prev = carry[0]
            acc = list(carry[1:])
            dle = plsc.load_gather(
                dl_c, [jnp.broadcast_to(q * 16, (16,)).astype(jnp.int32)])
            for e in range(16):
                row = q * 16 + e
                dnx = plsc.load_gather(
                    dl_c,
                    [jnp.broadcast_to(row + 1, (16,)).astype(jnp.int32)])
                fresh = dle != prev
                end = dle != dnx
                addr = dle * 16 + lane
                for j in range(D // 16):
                    msg = rows_b[row, pl.ds(j * 16, 16)]
                    acc[j] = jnp.where(fresh, msg,
                                       jnp.maximum(acc[j], msg))
                for j in range(D // 16):
                    plsc.store_scatter(aggr8[j], [addr], acc[j], mask=end)
                prev = dle
                dle = dnx
            return (prev, *acc)

        return plsc.parallel_loop(0, GC // 16, carry=carry)(qbody)

    def body(g, carry):
        def work(bc, bn, carry):
            idx_c, dl_c, rows_c, sem_c = bc
            idx_n, dl_n, rows_n, sem_n = bn
            ok_next = g + 1 < nch

            @pl.when(ok_next)
            def _():
                pltpu.sync_copy(ssrc.at[pl.ds(base + (g + 1) * GC, GC)],
                                idx_n)
                pltpu.sync_copy(sdlo.at[pl.ds(base + (g + 1) * GC, GC)],
                                dl_n.at[pl.ds(0, GC)])
                pltpu.async_copy(h.at[idx_n], rows_n, sem_n)

            pltpu.make_async_copy(h.at[idx_c], rows_c, sem_c).wait()
            return compute(dl_c, rows_c, ok_next, dl_n, carry)

        return lax.cond(jnp.bitwise_and(g, 1) == 0,
                        lambda c: work(bufs[0], bufs[1], c),
                        lambda c: work(bufs[1], bufs[0], c),
                        carry)

    carry0 = (jnp.full((16,), -1, jnp.int32),) + (neg_inf,) * (D // 16)
    lax.fori_loop(0, nch, body, carry0)

    nmb = jnp.where(w == NW - 1, LAST // 40, RANGE // 40)

    def chunk_out(c, _):
        def fin_body(i, _):
            for j, a in enumerate(aggr8):
                merged[i, pl.ds(j * 16, 16)] = a[
                    pl.ds((c * 40 + i) * 16, 16)]
            return 0

        lax.fori_loop(0, 40, fin_body, 0)
        pltpu.sync_copy(merged, out.at[pl.ds(lo + c * 40, 40)])
        return 0

    lax.fori_loop(0, nmb, chunk_out, 0)


def _pre_body(x_ref, w_ref, b_ref, o_ref):
    o_ref[...] = (
        jnp.dot(x_ref[...], w_ref[...], preferred_element_type=jnp.float32)
        + b_ref[...]
    )


def _conv_body(act, a_ref, h_ref, wl_ref, wr_ref, b_ref, o_ref):
    y = (
        jnp.dot(a_ref[...], wl_ref[...], preferred_element_type=jnp.float32)
        + jnp.dot(h_ref[...], wr_ref[...], preferred_element_type=jnp.float32)
        + b_ref[...]
    )
    if act == "relu":
        y = jnp.maximum(y, 0.0)
    elif act == "l2":
        n = jnp.sqrt(jnp.sum(y * y, axis=-1, keepdims=True))
        y = y / jnp.maximum(n, 1e-12)
    o_ref[...] = y


def _pre(x, wt, b):
    return pl.pallas_call(
        _pre_body,
        out_shape=jax.ShapeDtypeStruct((N, D), jnp.float32),
    )(x, wt, b.reshape(1, D))


def _conv(act, aggr, h, wlt, wrt, b):
    return pl.pallas_call(
        functools.partial(_conv_body, act),
        out_shape=jax.ShapeDtypeStruct((N, D), jnp.float32),
    )(aggr, h, wlt, wrt, b.reshape(1, D))


def kernel(x, edge_index, W_pre, b_pre, W_l_first, b_l_first, W_r_first,
           W_l_h0, b_l_h0, W_r_h0, W_l_h1, b_l_h1, W_r_h1,
           W_l_h2, b_l_h2, W_r_h2, W_l_out, b_l_out, W_r_out):
    _, _, counts, ssrc, sdlo = _plan(edge_index)

    wp = W_pre.T
    wlf, wrf = W_l_first.T, W_r_first.T
    wlo, wro = W_l_out.T, W_r_out.T
    hidden = [(W_l_h0.T, b_l_h0, W_r_h0.T),
              (W_l_h1.T, b_l_h1, W_r_h1.T),
              (W_l_h2.T, b_l_h2, W_r_h2.T)]

    outs = []
    h = x
    for i in range(3):
        h = _pre(h, wp, b_pre)
        a = _aggregate(h, ssrc, sdlo, counts)
        h = _conv("relu", a, h, wlf, wrf, b_l_first)
        wlh, blh, wrh = hidden[i]
        a = _aggregate(h, ssrc, sdlo, counts)
        h = _conv("relu", a, h, wlh, wrh, blh)
        a = _aggregate(h, ssrc, sdlo, counts)
        h = _conv("l2", a, h, wlo, wro, b_l_out)
        outs.append(h)
    return tuple(outs)
